# baseline (speedup 1.0000x reference)
import functools

import jax
import jax.numpy as jnp
from jax import lax
from jax.experimental import pallas as pl
from jax.experimental.pallas import tpu as pltpu
from jax.experimental.pallas import tpu_sc as plsc

IDXW = 128
CM = 2
NBUF = 3
AHEAD = 1


@functools.lru_cache(maxsize=None)
def _build_gather(B, V, D):
    info = plsc.get_sparse_core_info()
    NC, NS = info.num_cores, info.num_subcores
    NW = NC * NS
    chunk = IDXW * CM
    assert B % (NW * chunk) == 0
    b_per_w = B // NW
    n = b_per_w // chunk
    r = n % NBUF
    assert 1 <= AHEAD <= min(r, NBUF) and n > NBUF + r
    mesh = plsc.VectorSubcoreMesh(core_axis_name="c", subcore_axis_name="s")

    @functools.partial(
        pl.kernel,
        mesh=mesh,
        out_type=jax.ShapeDtypeStruct((B, D), jnp.float32),
        scratch_types=[
            pltpu.VMEM((n * CM, IDXW), jnp.int32),
        ]
        + [pltpu.VMEM((chunk, D), jnp.float32)] * NBUF
        + [pltpu.SemaphoreType.DMA] * (2 * NBUF),
    )
    def gather_kernel(idx_hbm, table_hbm, out_hbm, idx_v, *bufs_sems):
        bufs = bufs_sems[:NBUF]
        gsem = bufs_sems[NBUF : 2 * NBUF]
        osem = bufs_sems[2 * NBUF :]
        wid = lax.axis_index("s") * NC + lax.axis_index("c")
        base = wid * b_per_w
        pltpu.sync_copy(idx_hbm.at[wid], idx_v)

        def g_copies(j, b):
            return [
                pltpu.make_async_copy(
                    table_hbm.at[idx_v.at[j * CM + m]],
                    bufs[b].at[pl.ds(m * IDXW, IDXW)],
                    gsem[b],
                )
                for m in range(CM)
            ]

        def start_g(j, b):
            for m in range(CM):
                pltpu.async_copy(
                    table_hbm.at[idx_v.at[j * CM + m]],
                    bufs[b].at[pl.ds(m * IDXW, IDXW)],
                    gsem[b],
                )

        def wait_g(j, b):
            for c in g_copies(j, b):
                c.wait()

        def o_copy(j, b):
            return pltpu.make_async_copy(
                bufs[b], out_hbm.at[pl.ds(base + j * chunk, chunk)], osem[b]
            )

        for j in range(AHEAD):
            start_g(j, j % NBUF)

        def flat_iter(i, b, do_issue, issue_owait):
            if do_issue:
                j = i + AHEAD
                bj = (b + AHEAD) % NBUF
                if issue_owait:
                    o_copy(j - NBUF, bj).wait()
                start_g(j, bj)
            wait_g(i, b)
            pltpu.async_copy(
                bufs[b], out_hbm.at[pl.ds(base + i * chunk, chunk)], osem[b]
            )

        for i in range(NBUF):
            j = i + AHEAD
            flat_iter(i, i % NBUF, j < n, j >= NBUF)

        def body(g, _):
            for b in range(NBUF):
                i = g * NBUF + b
                flat_iter(i, b, True, True)
            return 0

        lax.fori_loop(1, n // NBUF, body, 0)

        for i in range(n - r, n):
            j = i + AHEAD
            flat_iter(i, i % NBUF, j < n, j >= NBUF)

        for k in range(n - NBUF, n):
            o_copy(k, k % NBUF).wait()

    return gather_kernel


def kernel(input_ids, embedding_table):
    lead_shape = input_ids.shape
    idx = input_ids.reshape(-1).astype(jnp.int32)
    B = idx.shape[0]
    V, D = embedding_table.shape
    info = plsc.get_sparse_core_info()
    NW = info.num_cores * info.num_subcores
    idx3 = idx.reshape(NW, B // (NW * IDXW), IDXW)
    out = _build_gather(B, V, D)(idx3, embedding_table)
    return out.reshape(*lead_shape, D)


# speedup vs baseline: 1.0111x; 1.0111x over previous
import functools

import jax
import jax.numpy as jnp
from jax import lax
from jax.experimental import pallas as pl
from jax.experimental.pallas import tpu as pltpu
from jax.experimental.pallas import tpu_sc as plsc

IDXW = 128
CM = 1
NBUF = 6
AHEAD = 2


@functools.lru_cache(maxsize=None)
def _build_gather(B, V, D):
    info = plsc.get_sparse_core_info()
    NC, NS = info.num_cores, info.num_subcores
    NW = NC * NS
    chunk = IDXW * CM
    assert B % (NW * chunk) == 0
    b_per_w = B // NW
    n = b_per_w // chunk
    r = n % NBUF
    assert 1 <= AHEAD <= min(r, NBUF) and n > NBUF + r
    mesh = plsc.VectorSubcoreMesh(core_axis_name="c", subcore_axis_name="s")

    @functools.partial(
        pl.kernel,
        mesh=mesh,
        out_type=jax.ShapeDtypeStruct((B, D), jnp.float32),
        scratch_types=[
            pltpu.VMEM((n * CM, IDXW), jnp.int32),
        ]
        + [pltpu.VMEM((chunk, D), jnp.float32)] * NBUF
        + [pltpu.SemaphoreType.DMA] * (2 * NBUF),
    )
    def gather_kernel(idx_hbm, table_hbm, out_hbm, idx_v, *bufs_sems):
        bufs = bufs_sems[:NBUF]
        gsem = bufs_sems[NBUF : 2 * NBUF]
        osem = bufs_sems[2 * NBUF :]
        wid = lax.axis_index("s") * NC + lax.axis_index("c")
        base = wid * b_per_w
        pltpu.sync_copy(idx_hbm.at[wid], idx_v)

        def g_copies(j, b):
            return [
                pltpu.make_async_copy(
                    table_hbm.at[idx_v.at[j * CM + m]],
                    bufs[b].at[pl.ds(m * IDXW, IDXW)],
                    gsem[b],
                )
                for m in range(CM)
            ]

        def start_g(j, b):
            for m in range(CM):
                pltpu.async_copy(
                    table_hbm.at[idx_v.at[j * CM + m]],
                    bufs[b].at[pl.ds(m * IDXW, IDXW)],
                    gsem[b],
                )

        def wait_g(j, b):
            for c in g_copies(j, b):
                c.wait()

        def o_copy(j, b):
            return pltpu.make_async_copy(
                bufs[b], out_hbm.at[pl.ds(base + j * chunk, chunk)], osem[b]
            )

        for j in range(AHEAD):
            start_g(j, j % NBUF)

        def flat_iter(i, b, do_issue, issue_owait):
            if do_issue:
                j = i + AHEAD
                bj = (b + AHEAD) % NBUF
                if issue_owait:
                    o_copy(j - NBUF, bj).wait()
                start_g(j, bj)
            wait_g(i, b)
            pltpu.async_copy(
                bufs[b], out_hbm.at[pl.ds(base + i * chunk, chunk)], osem[b]
            )

        for i in range(NBUF):
            j = i + AHEAD
            flat_iter(i, i % NBUF, j < n, j >= NBUF)

        def body(g, _):
            for b in range(NBUF):
                i = g * NBUF + b
                flat_iter(i, b, True, True)
            return 0

        lax.fori_loop(1, n // NBUF, body, 0)

        for i in range(n - r, n):
            j = i + AHEAD
            flat_iter(i, i % NBUF, j < n, j >= NBUF)

        for k in range(n - NBUF, n):
            o_copy(k, k % NBUF).wait()

    return gather_kernel


def kernel(input_ids, embedding_table):
    lead_shape = input_ids.shape
    idx = input_ids.reshape(-1).astype(jnp.int32)
    B = idx.shape[0]
    V, D = embedding_table.shape
    info = plsc.get_sparse_core_info()
    NW = info.num_cores * info.num_subcores
    idx3 = idx.reshape(NW, B // (NW * IDXW), IDXW)
    out = _build_gather(B, V, D)(idx3, embedding_table)
    return out.reshape(*lead_shape, D)
